# baseline (speedup 1.0000x reference)
import functools

import jax
import jax.numpy as jnp
from jax import lax
from jax.experimental import pallas as pl
from jax.experimental.pallas import tpu as pltpu
from jax.experimental.pallas import tpu_sc as plsc

V, H, E = 32, 1024, 1536
N = 64
ROW = H * E
R = 256
D = ROW // R
K = 8
NW = 32
B_TOTAL = N * R
B_W = B_TOTAL // NW
N_CHUNK = B_W // K

_mesh = plsc.VectorSubcoreMesh(core_axis_name="c", subcore_axis_name="s")


@functools.partial(
    pl.kernel,
    mesh=_mesh,
    out_type=jax.ShapeDtypeStruct((B_TOTAL, D), jnp.float32),
    scratch_types=[
        pltpu.VMEM((B_W,), jnp.int32),
        pltpu.VMEM((K, D), jnp.float32),
        pltpu.SemaphoreType.DMA,
    ],
)
def _sc_gather(table_hbm, idx_hbm, out_hbm, idx_v, buf, sem):
    wid = lax.axis_index("s") * 2 + lax.axis_index("c")
    base = wid * B_W
    pltpu.sync_copy(idx_hbm.at[pl.ds(base, B_W)], idx_v)

    def body(j, carry):
        off = j * K
        pltpu.async_copy(table_hbm.at[idx_v.at[pl.ds(off, K)]], buf, sem).wait()
        pltpu.sync_copy(buf, out_hbm.at[pl.ds(base + off, K)])
        return carry

    lax.fori_loop(0, N_CHUNK, body, 0)


def kernel(cat_ids, W):
    table = W.reshape(V * R, D)
    idx = (cat_ids.astype(jnp.int32)[:, None] * R
           + jnp.arange(R, dtype=jnp.int32)[None, :]).reshape(-1)
    out2 = _sc_gather(table, idx)
    return out2.reshape(N, H, E)


# speedup vs baseline: 1.0381x; 1.0381x over previous
import functools

import jax
import jax.numpy as jnp
from jax import lax
from jax.experimental import pallas as pl
from jax.experimental.pallas import tpu as pltpu
from jax.experimental.pallas import tpu_sc as plsc

V, H, E = 32, 1024, 1536
N = 64
ROW = H * E
R = 256
D = ROW // R
K = 8
NW = 32
B_TOTAL = N * R
B_W = B_TOTAL // NW
N_CHUNK = B_W // K

_mesh = plsc.VectorSubcoreMesh(core_axis_name="c", subcore_axis_name="s")


NP = N_CHUNK // 2


@functools.partial(
    pl.kernel,
    mesh=_mesh,
    out_type=jax.ShapeDtypeStruct((B_TOTAL, D), jnp.float32),
    scratch_types=[
        pltpu.VMEM((B_W,), jnp.int32),
        pltpu.VMEM((K, D), jnp.float32),
        pltpu.VMEM((K, D), jnp.float32),
        pltpu.SemaphoreType.DMA,
        pltpu.SemaphoreType.DMA,
        pltpu.SemaphoreType.DMA,
        pltpu.SemaphoreType.DMA,
    ],
)
def _sc_gather(table_hbm, idx_hbm, out_hbm, idx_v, buf0, buf1, g0, g1, w0, w1):
    wid = lax.axis_index("s") * 2 + lax.axis_index("c")
    base = wid * B_W
    pltpu.sync_copy(idx_hbm.at[pl.ds(base, B_W)], idx_v)

    def gather(j, buf, sem):
        pltpu.async_copy(table_hbm.at[idx_v.at[pl.ds(j * K, K)]], buf, sem)

    def write(j, buf, sem):
        pltpu.async_copy(buf, out_hbm.at[pl.ds(base + j * K, K)], sem)

    def wait_gather(buf, sem):
        pltpu.make_async_copy(table_hbm.at[pl.ds(0, K)], buf, sem).wait()

    def wait_write(buf, sem):
        pltpu.make_async_copy(buf, out_hbm.at[pl.ds(base, K)], sem).wait()

    gather(0, buf0, g0)

    def body(p, carry):
        j0 = 2 * p
        wait_gather(buf0, g0)
        write(j0, buf0, w0)

        @pl.when(p > 0)
        def _():
            wait_write(buf1, w1)

        gather(j0 + 1, buf1, g1)

        wait_gather(buf1, g1)
        write(j0 + 1, buf1, w1)
        wait_write(buf0, w0)

        @pl.when(p < NP - 1)
        def _():
            gather(j0 + 2, buf0, g0)

        return carry

    lax.fori_loop(0, NP, body, 0)
    wait_write(buf1, w1)


def kernel(cat_ids, W):
    table = W.reshape(V * R, D)
    idx = (cat_ids.astype(jnp.int32)[:, None] * R
           + jnp.arange(R, dtype=jnp.int32)[None, :]).reshape(-1)
    out2 = _sc_gather(table, idx)
    return out2.reshape(N, H, E)


# speedup vs baseline: 1.0746x; 1.0352x over previous
import functools

import jax
import jax.numpy as jnp
from jax import lax
from jax.experimental import pallas as pl
from jax.experimental.pallas import tpu as pltpu
from jax.experimental.pallas import tpu_sc as plsc

V, H, E = 32, 1024, 1536
N = 64
ROW = H * E
R = 32
D = ROW // R
NW = 32
B_TOTAL = N * R
B_W = B_TOTAL // NW
NP = B_W // 2

_mesh = plsc.VectorSubcoreMesh(core_axis_name="c", subcore_axis_name="s")


@functools.partial(
    pl.kernel,
    mesh=_mesh,
    out_type=jax.ShapeDtypeStruct((B_TOTAL, D), jnp.float32),
    scratch_types=[
        pltpu.VMEM((B_W, 8), jnp.int32),
        pltpu.VMEM((1, D), jnp.float32),
        pltpu.VMEM((1, D), jnp.float32),
        pltpu.SemaphoreType.DMA,
        pltpu.SemaphoreType.DMA,
        pltpu.SemaphoreType.DMA,
        pltpu.SemaphoreType.DMA,
    ],
)
def _sc_gather(table_hbm, idx_hbm, out_hbm, idx_v, buf0, buf1, g0, g1, w0, w1):
    wid = lax.axis_index("s") * 2 + lax.axis_index("c")
    base = wid * B_W
    pltpu.sync_copy(idx_hbm.at[pl.ds(base, B_W)], idx_v)

    def gather(j, buf, sem):
        pltpu.async_copy(table_hbm.at[idx_v.at[j, pl.ds(0, 1)]], buf, sem)

    def write(j, buf, sem):
        pltpu.async_copy(buf, out_hbm.at[pl.ds(base + j, 1)], sem)

    def wait_gather(buf, sem):
        pltpu.make_async_copy(table_hbm.at[pl.ds(0, 1)], buf, sem).wait()

    def wait_write(buf, sem):
        pltpu.make_async_copy(buf, out_hbm.at[pl.ds(base, 1)], sem).wait()

    gather(0, buf0, g0)

    def body(p, carry):
        j0 = 2 * p
        wait_gather(buf0, g0)
        write(j0, buf0, w0)

        @pl.when(p > 0)
        def _():
            wait_write(buf1, w1)

        gather(j0 + 1, buf1, g1)

        wait_gather(buf1, g1)
        write(j0 + 1, buf1, w1)
        wait_write(buf0, w0)

        @pl.when(p < NP - 1)
        def _():
            gather(j0 + 2, buf0, g0)

        return carry

    lax.fori_loop(0, NP, body, 0)
    wait_write(buf1, w1)


def kernel(cat_ids, W):
    table = W.reshape(V * R, D)
    idx = (cat_ids.astype(jnp.int32)[:, None] * R
           + jnp.arange(R, dtype=jnp.int32)[None, :]).reshape(-1)
    idx8 = jnp.broadcast_to(idx[:, None], (B_TOTAL, 8))
    out2 = _sc_gather(table, idx8)
    return out2.reshape(N, H, E)
